# baseline (device time: 118705 ns/iter reference)
import jax
import jax.numpy as jnp
from jax import lax
from jax.experimental import pallas as pl
from jax.experimental.pallas import tpu as pltpu

N_DEV = 32
M = 2048
N = 2048
CHUNK = M // N_DEV

MASKS_A = (1, 8, 2, 16, 4)
DUALS_A = MASKS_A
MASKS_B = (8, 1, 16, 4, 2)
DUALS_B = MASKS_B
MASKS_C = (3, 4, 2, 8, 16)
DUALS_C = (1, 4, 3, 8, 16)
COLS = (768, 640, 640)
COL_OFF = (0, 768, 1408)

RS_SIZE = (16, 8, 4, 2, 1)
RS_OFF = (0, 16, 24, 28, 30)


def _parity(x):
    t = x ^ (x >> 1)
    t = t ^ (t >> 2)
    t = t ^ (t >> 4)
    return t & 1


def _rowblock(v, masks):
    p = 0
    for j, m in enumerate(masks):
        p = p ^ (m * ((v >> (4 - j)) & 1))
    return p


def kernel(A, B):
    a16 = A.astype(jnp.bfloat16)
    b16 = B.astype(jnp.bfloat16)

    def body(a_ref, b_ref, out_ref,
             pa_ref, pb_ref, pc_ref, rsa_ref, rsb_ref, rsc_ref,
             gata_ref, gatb_ref, gatc_ref, send_sems, recv_sems):
        my = lax.axis_index("i")

        def virt(duals):
            u = 0
            for j, d in enumerate(duals):
                u = u | (_parity(my & d) << (4 - j))
            return u

        def rs_offsets(u):
            soff, nbase = [], []
            for j in range(5):
                half = RS_SIZE[j]
                mybit = (u >> (4 - j)) & 1
                base = (u >> (5 - j)) << (5 - j)
                soff.append(base + (1 - mybit) * half)
                nbase.append(base + mybit * half)
            return soff, nbase

        class Part:
            def __init__(self, masks, duals, p_ref, rs_ref, gat_ref,
                         col_off, cols, sem_off):
                self.masks = masks
                self.u = virt(duals)
                self.partners = [my ^ m for m in masks]
                self.p_ref = p_ref
                self.rs_ref = rs_ref
                self.gat_ref = gat_ref
                self.col_off = col_off
                self.cols = cols
                self.sem_off = sem_off
                self.soff, self.nbase = rs_offsets(self.u)
                self.rd = None

        parts = [
            Part(MASKS_A, DUALS_A, pa_ref, rsa_ref, gata_ref,
                 COL_OFF[0], COLS[0], 0),
            Part(MASKS_B, DUALS_B, pb_ref, rsb_ref, gatb_ref,
                 COL_OFF[1], COLS[1], 10),
            Part(MASKS_C, DUALS_C, pc_ref, rsc_ref, gatc_ref,
                 COL_OFF[2], COLS[2], 20),
        ]

        barrier = pltpu.get_barrier_semaphore()
        for mk in (1, 2, 3, 4, 8, 16):
            pl.semaphore_signal(barrier, inc=1, device_id=(my ^ mk,),
                                device_id_type=pl.DeviceIdType.MESH)
        pl.semaphore_wait(barrier, 6)

        def matmul_half(pt, vbase):
            for i in range(2):
                v0 = vbase + 8 * i
                ablk = jnp.concatenate(
                    [
                        a_ref[pl.ds(_rowblock(v0 + t, pt.masks) * CHUNK,
                                    CHUNK), :]
                        for t in range(8)
                    ],
                    axis=0,
                )
                pt.p_ref[pl.ds(v0 * CHUNK, 8 * CHUNK), :] = jnp.dot(
                    ablk, b_ref[:, pl.ds(pt.col_off, pt.cols)],
                    preferred_element_type=jnp.float32,
                ).astype(jnp.bfloat16)

        def rs_rdma(pt, j):
            half = RS_SIZE[j]
            return pltpu.make_async_remote_copy(
                src_ref=pt.p_ref.at[pl.ds(pt.soff[j] * CHUNK,
                                          half * CHUNK), :],
                dst_ref=pt.rs_ref.at[pl.ds(RS_OFF[j] * CHUNK,
                                           half * CHUNK), :],
                send_sem=send_sems.at[pt.sem_off + j],
                recv_sem=recv_sems.at[pt.sem_off + j],
                device_id=(pt.partners[j],),
                device_id_type=pl.DeviceIdType.MESH,
            )

        def acc_sub(pt, j, x, length):
            dst = pl.ds(x * CHUNK, length * CHUNK)
            src = pl.ds((RS_OFF[j] + x - pt.nbase[j]) * CHUNK,
                        length * CHUNK)
            pt.p_ref[dst, :] = pt.p_ref[dst, :] + pt.rs_ref[src, :]

        def ag_rdma(pt, j):
            size = 1 << j
            cb = (pt.u >> j) << j
            blk_sl = pl.ds(cb * CHUNK, size * CHUNK)
            return pltpu.make_async_remote_copy(
                src_ref=pt.gat_ref.at[blk_sl, :],
                dst_ref=pt.gat_ref.at[blk_sl, :],
                send_sem=send_sems.at[pt.sem_off + 5 + j],
                recv_sem=recv_sems.at[pt.sem_off + 5 + j],
                device_id=(pt.partners[4 - j],),
                device_id_type=pl.DeviceIdType.MESH,
            )

        def scatter(pt, vbase, n):
            for t in range(n):
                v = vbase + t
                out_ref[pl.ds(_rowblock(v, pt.masks) * CHUNK, CHUNK),
                        pl.ds(pt.col_off, pt.cols)] = (
                    pt.gat_ref[pl.ds(v * CHUNK, CHUNK), :]
                )

        for pt in parts:
            matmul_half(pt, pt.soff[0])
            pt.rd = rs_rdma(pt, 0)
            pt.rd.start()
        for pt in parts:
            matmul_half(pt, pt.nbase[0])

        for j in range(4):
            nh = RS_SIZE[j + 1]
            for pt in parts:
                pt.rd.wait_recv()
                acc_sub(pt, j, pt.soff[j + 1], nh)
                rd_n = rs_rdma(pt, j + 1)
                rd_n.start()
                pt.rd_prev, pt.rd = pt.rd, rd_n
            for pt in parts:
                acc_sub(pt, j, pt.nbase[j + 1], nh)
            for pt in parts:
                pt.rd_prev.wait_send()

        for pt in parts:
            pt.rd.wait_recv()
            acc_sub(pt, 4, pt.nbase[4], 1)
            mine = pl.ds(pt.u * CHUNK, CHUNK)
            pt.gat_ref[mine, :] = jnp.maximum(pt.p_ref[mine, :], 0)
            rd_n = ag_rdma(pt, 0)
            rd_n.start()
            pt.rd_prev, pt.rd = pt.rd, rd_n
        for pt in parts:
            pt.rd_prev.wait_send()

        for j in range(5):
            for pt in parts:
                if j == 0:
                    scatter(pt, pt.u, 1)
                else:
                    half = 1 << (j - 1)
                    scatter(pt, ((pt.u >> (j - 1)) << (j - 1)) ^ half, half)
            for pt in parts:
                pt.rd.wait_recv()
                if j < 4:
                    rd_n = ag_rdma(pt, j + 1)
                    rd_n.start()
                    pt.rd_prev, pt.rd = pt.rd, rd_n
                else:
                    pt.rd_prev = pt.rd
            for pt in parts:
                pt.rd_prev.wait_send()

        for pt in parts:
            scatter(pt, ((pt.u >> 4) << 4) ^ 16, 16)

    return pl.pallas_call(
        body,
        out_shape=jax.ShapeDtypeStruct((M, N), jnp.bfloat16),
        in_specs=[
            pl.BlockSpec(memory_space=pltpu.VMEM),
            pl.BlockSpec(memory_space=pltpu.VMEM),
        ],
        out_specs=pl.BlockSpec(memory_space=pltpu.VMEM),
        scratch_shapes=[
            pltpu.VMEM((M, COLS[0]), jnp.bfloat16),
            pltpu.VMEM((M, COLS[1]), jnp.bfloat16),
            pltpu.VMEM((M, COLS[2]), jnp.bfloat16),
            pltpu.VMEM((31 * CHUNK, COLS[0]), jnp.bfloat16),
            pltpu.VMEM((31 * CHUNK, COLS[1]), jnp.bfloat16),
            pltpu.VMEM((31 * CHUNK, COLS[2]), jnp.bfloat16),
            pltpu.VMEM((M, COLS[0]), jnp.bfloat16),
            pltpu.VMEM((M, COLS[1]), jnp.bfloat16),
            pltpu.VMEM((M, COLS[2]), jnp.bfloat16),
            pltpu.SemaphoreType.DMA((30,)),
            pltpu.SemaphoreType.DMA((30,)),
        ],
        compiler_params=pltpu.CompilerParams(collective_id=0),
    )(a16, b16)


# device time: 116546 ns/iter; 1.0185x vs baseline; 1.0185x over previous
import jax
import jax.numpy as jnp
from jax import lax
from jax.experimental import pallas as pl
from jax.experimental.pallas import tpu as pltpu

N_DEV = 32
M = 2048
N = 2048
CHUNK = M // N_DEV

MASKS_A = (1, 8, 2, 16, 4)
DUALS_A = MASKS_A
MASKS_B = (8, 1, 16, 4, 2)
DUALS_B = MASKS_B
MASKS_C = (3, 4, 2, 8, 16)
DUALS_C = (1, 4, 3, 8, 16)
COLS = (768, 768, 512)
COL_OFF = (0, 768, 1536)

RS_SIZE = (16, 8, 4, 2, 1)
RS_OFF = (0, 16, 24, 28, 30)


def _parity(x):
    t = x ^ (x >> 1)
    t = t ^ (t >> 2)
    t = t ^ (t >> 4)
    return t & 1


def _rowblock(v, masks):
    p = 0
    for j, m in enumerate(masks):
        p = p ^ (m * ((v >> (4 - j)) & 1))
    return p


def kernel(A, B):
    a16 = A.astype(jnp.bfloat16)
    b16 = B.astype(jnp.bfloat16)

    def body(a_ref, b_ref, out_ref,
             pa_ref, pb_ref, pc_ref, rsa_ref, rsb_ref, rsc_ref,
             gata_ref, gatb_ref, gatc_ref, send_sems, recv_sems):
        my = lax.axis_index("i")

        def virt(duals):
            u = 0
            for j, d in enumerate(duals):
                u = u | (_parity(my & d) << (4 - j))
            return u

        def rs_offsets(u):
            soff, nbase = [], []
            for j in range(5):
                half = RS_SIZE[j]
                mybit = (u >> (4 - j)) & 1
                base = (u >> (5 - j)) << (5 - j)
                soff.append(base + (1 - mybit) * half)
                nbase.append(base + mybit * half)
            return soff, nbase

        class Part:
            def __init__(self, masks, duals, p_ref, rs_ref, gat_ref,
                         col_off, cols, sem_off):
                self.masks = masks
                self.u = virt(duals)
                self.partners = [my ^ m for m in masks]
                self.p_ref = p_ref
                self.rs_ref = rs_ref
                self.gat_ref = gat_ref
                self.col_off = col_off
                self.cols = cols
                self.sem_off = sem_off
                self.soff, self.nbase = rs_offsets(self.u)
                self.rd = None

        parts = [
            Part(MASKS_A, DUALS_A, pa_ref, rsa_ref, gata_ref,
                 COL_OFF[0], COLS[0], 0),
            Part(MASKS_B, DUALS_B, pb_ref, rsb_ref, gatb_ref,
                 COL_OFF[1], COLS[1], 10),
            Part(MASKS_C, DUALS_C, pc_ref, rsc_ref, gatc_ref,
                 COL_OFF[2], COLS[2], 20),
        ]

        barrier = pltpu.get_barrier_semaphore()
        for mk in (1, 2, 3, 4, 8, 16):
            pl.semaphore_signal(barrier, inc=1, device_id=(my ^ mk,),
                                device_id_type=pl.DeviceIdType.MESH)
        pl.semaphore_wait(barrier, 6)

        def matmul_half(pt, vbase):
            for i in range(2):
                v0 = vbase + 8 * i
                ablk = jnp.concatenate(
                    [
                        a_ref[pl.ds(_rowblock(v0 + t, pt.masks) * CHUNK,
                                    CHUNK), :]
                        for t in range(8)
                    ],
                    axis=0,
                )
                pt.p_ref[pl.ds(v0 * CHUNK, 8 * CHUNK), :] = jnp.dot(
                    ablk, b_ref[:, pl.ds(pt.col_off, pt.cols)],
                    preferred_element_type=jnp.float32,
                ).astype(jnp.bfloat16)

        def rs_rdma(pt, j):
            half = RS_SIZE[j]
            return pltpu.make_async_remote_copy(
                src_ref=pt.p_ref.at[pl.ds(pt.soff[j] * CHUNK,
                                          half * CHUNK), :],
                dst_ref=pt.rs_ref.at[pl.ds(RS_OFF[j] * CHUNK,
                                           half * CHUNK), :],
                send_sem=send_sems.at[pt.sem_off + j],
                recv_sem=recv_sems.at[pt.sem_off + j],
                device_id=(pt.partners[j],),
                device_id_type=pl.DeviceIdType.MESH,
            )

        def acc_sub(pt, j, x, length):
            dst = pl.ds(x * CHUNK, length * CHUNK)
            src = pl.ds((RS_OFF[j] + x - pt.nbase[j]) * CHUNK,
                        length * CHUNK)
            pt.p_ref[dst, :] = pt.p_ref[dst, :] + pt.rs_ref[src, :]

        def ag_rdma(pt, j):
            size = 1 << j
            cb = (pt.u >> j) << j
            blk_sl = pl.ds(cb * CHUNK, size * CHUNK)
            return pltpu.make_async_remote_copy(
                src_ref=pt.gat_ref.at[blk_sl, :],
                dst_ref=pt.gat_ref.at[blk_sl, :],
                send_sem=send_sems.at[pt.sem_off + 5 + j],
                recv_sem=recv_sems.at[pt.sem_off + 5 + j],
                device_id=(pt.partners[4 - j],),
                device_id_type=pl.DeviceIdType.MESH,
            )

        def scatter(pt, vbase, n):
            for t in range(n):
                v = vbase + t
                out_ref[pl.ds(_rowblock(v, pt.masks) * CHUNK, CHUNK),
                        pl.ds(pt.col_off, pt.cols)] = (
                    pt.gat_ref[pl.ds(v * CHUNK, CHUNK), :]
                )

        for pt in parts:
            matmul_half(pt, pt.soff[0])
            pt.rd = rs_rdma(pt, 0)
            pt.rd.start()
        for pt in parts:
            matmul_half(pt, pt.nbase[0])

        for j in range(4):
            nh = RS_SIZE[j + 1]
            for pt in parts:
                pt.rd.wait_recv()
                acc_sub(pt, j, pt.soff[j + 1], nh)
                rd_n = rs_rdma(pt, j + 1)
                rd_n.start()
                pt.rd_prev, pt.rd = pt.rd, rd_n
            for pt in parts:
                acc_sub(pt, j, pt.nbase[j + 1], nh)
            for pt in parts:
                pt.rd_prev.wait_send()

        for pt in parts:
            pt.rd.wait_recv()
            acc_sub(pt, 4, pt.nbase[4], 1)
            mine = pl.ds(pt.u * CHUNK, CHUNK)
            pt.gat_ref[mine, :] = jnp.maximum(pt.p_ref[mine, :], 0)
            rd_n = ag_rdma(pt, 0)
            rd_n.start()
            pt.rd_prev, pt.rd = pt.rd, rd_n
        for pt in parts:
            pt.rd_prev.wait_send()

        for j in range(5):
            for pt in parts:
                if j == 0:
                    scatter(pt, pt.u, 1)
                else:
                    half = 1 << (j - 1)
                    scatter(pt, ((pt.u >> (j - 1)) << (j - 1)) ^ half, half)
            for pt in parts:
                pt.rd.wait_recv()
                if j < 4:
                    rd_n = ag_rdma(pt, j + 1)
                    rd_n.start()
                    pt.rd_prev, pt.rd = pt.rd, rd_n
                else:
                    pt.rd_prev = pt.rd
            for pt in parts:
                pt.rd_prev.wait_send()

        for pt in parts:
            scatter(pt, ((pt.u >> 4) << 4) ^ 16, 16)

    return pl.pallas_call(
        body,
        out_shape=jax.ShapeDtypeStruct((M, N), jnp.bfloat16),
        in_specs=[
            pl.BlockSpec(memory_space=pltpu.VMEM),
            pl.BlockSpec(memory_space=pltpu.VMEM),
        ],
        out_specs=pl.BlockSpec(memory_space=pltpu.VMEM),
        scratch_shapes=[
            pltpu.VMEM((M, COLS[0]), jnp.bfloat16),
            pltpu.VMEM((M, COLS[1]), jnp.bfloat16),
            pltpu.VMEM((M, COLS[2]), jnp.bfloat16),
            pltpu.VMEM((31 * CHUNK, COLS[0]), jnp.bfloat16),
            pltpu.VMEM((31 * CHUNK, COLS[1]), jnp.bfloat16),
            pltpu.VMEM((31 * CHUNK, COLS[2]), jnp.bfloat16),
            pltpu.VMEM((M, COLS[0]), jnp.bfloat16),
            pltpu.VMEM((M, COLS[1]), jnp.bfloat16),
            pltpu.VMEM((M, COLS[2]), jnp.bfloat16),
            pltpu.SemaphoreType.DMA((30,)),
            pltpu.SemaphoreType.DMA((30,)),
        ],
        compiler_params=pltpu.CompilerParams(collective_id=0),
    )(a16, b16)
